# Initial kernel scaffold; baseline (speedup 1.0000x reference)
import jax
import jax.numpy as jnp
from jax.experimental import pallas as pl


def kernel(atom_fea, target_index):
    raise NotImplementedError("write your pallas kernel here")


# speedup vs baseline: 1.2446x; 1.2446x over previous
import functools

import jax
import jax.numpy as jnp
from jax import lax
from jax.experimental import pallas as pl
from jax.experimental.pallas import tpu as pltpu
from jax.experimental.pallas import tpu_sc as plsc

B = 64
N = 4096
N0 = 1024
D = 128

NC = 2
NS = 16
NW = NC * NS

ROWS_TOTAL = B * N0
ROWS_PER_W = ROWS_TOTAL // NW
CHUNK = 128
NCHUNK = ROWS_PER_W // CHUNK
LANES = 16


def _body(table_hbm, idx_hbm, out_hbm, idx_v, rows_v, sem):
    cid = lax.axis_index("c")
    sid = lax.axis_index("s")
    wid = sid * NC + cid
    base_row = wid * ROWS_PER_W

    pltpu.sync_copy(idx_hbm.at[wid], idx_v)

    for c in range(NCHUNK):
        batch = wid * (ROWS_PER_W // N0) + (c * CHUNK) // N0
        off = batch * N
        for j in range(CHUNK // LANES):
            sl = pl.ds(j * LANES, LANES)
            idx_v[c, sl] = idx_v[c, sl] + off

    for c in range(NCHUNK):
        pltpu.async_copy(table_hbm.at[idx_v.at[c]], rows_v, sem).wait()
        pltpu.sync_copy(rows_v, out_hbm.at[pl.ds(base_row + c * CHUNK, CHUNK)])


@jax.jit
def _gather(table, idx):
    mesh = plsc.VectorSubcoreMesh(
        core_axis_name="c", subcore_axis_name="s",
        num_cores=NC, num_subcores=NS)
    return pl.kernel(
        _body,
        out_type=jax.ShapeDtypeStruct((ROWS_TOTAL, D), jnp.float32),
        mesh=mesh,
        scratch_types=[
            pltpu.VMEM((NCHUNK, CHUNK), jnp.int32),
            pltpu.VMEM((CHUNK, D), jnp.float32),
            pltpu.SemaphoreType.DMA,
        ],
    )(table, idx)


def kernel(atom_fea, target_index):
    table = atom_fea.reshape(B * N, D)
    idx = target_index.astype(jnp.int32).reshape(NW, NCHUNK, CHUNK)
    out = _gather(table, idx)
    return out.reshape(B, N0, D)


# speedup vs baseline: 1.5613x; 1.2545x over previous
import functools

import jax
import jax.numpy as jnp
from jax import lax
from jax.experimental import pallas as pl
from jax.experimental.pallas import tpu as pltpu
from jax.experimental.pallas import tpu_sc as plsc

B = 64
N = 4096
N0 = 1024
D = 128

NC = 2
NS = 16
NW = NC * NS

ROWS_TOTAL = B * N0
ROWS_PER_W = ROWS_TOTAL // NW
CHUNK = 128
NCHUNK = ROWS_PER_W // CHUNK
LANES = 16


NBUF = 6
LOOKAHEAD = 3


def _body(table_hbm, idx_hbm, out_hbm, idx_v, *rest):
    bufs = rest[:NBUF]
    gsems = rest[NBUF:2 * NBUF]
    wsems = rest[2 * NBUF:3 * NBUF]

    cid = lax.axis_index("c")
    sid = lax.axis_index("s")
    wid = sid * NC + cid
    base_row = wid * ROWS_PER_W

    pltpu.sync_copy(idx_hbm.at[wid], idx_v)

    for c in range(NCHUNK):
        batch = wid * (ROWS_PER_W // N0) + (c * CHUNK) // N0
        off = batch * N
        for j in range(CHUNK // LANES):
            sl = pl.ds(j * LANES, LANES)
            idx_v[c, sl] = idx_v[c, sl] + off

    def gather(c):
        b = c % NBUF
        return pltpu.async_copy(table_hbm.at[idx_v.at[c]], bufs[b], gsems[b])

    def writeout(c):
        b = c % NBUF
        return pltpu.async_copy(
            bufs[b], out_hbm.at[pl.ds(base_row + c * CHUNK, CHUNK)], wsems[b])

    gd = {c: gather(c) for c in range(min(LOOKAHEAD, NCHUNK))}
    wd = {}
    w_waited = set()
    for c in range(NCHUNK):
        gd[c].wait()
        wd[c] = writeout(c)
        f = c + LOOKAHEAD
        if f < NCHUNK:
            p = f - NBUF
            if p >= 0:
                wd[p].wait()
                w_waited.add(p)
            gd[f] = gather(f)
    for c in range(NCHUNK):
        if c not in w_waited:
            wd[c].wait()


@jax.jit
def _gather(table, idx):
    mesh = plsc.VectorSubcoreMesh(
        core_axis_name="c", subcore_axis_name="s",
        num_cores=NC, num_subcores=NS)
    return pl.kernel(
        _body,
        out_type=jax.ShapeDtypeStruct((ROWS_TOTAL, D), jnp.float32),
        mesh=mesh,
        scratch_types=(
            [pltpu.VMEM((NCHUNK, CHUNK), jnp.int32)]
            + [pltpu.VMEM((CHUNK, D), jnp.float32) for _ in range(NBUF)]
            + [pltpu.SemaphoreType.DMA for _ in range(2 * NBUF)]
        ),
    )(table, idx)


def kernel(atom_fea, target_index):
    table = atom_fea.reshape(B * N, D)
    idx = target_index.astype(jnp.int32).reshape(NW, NCHUNK, CHUNK)
    out = _gather(table, idx)
    return out.reshape(B, N0, D)
